# baseline (speedup 1.0000x reference)
import functools

import jax
import jax.numpy as jnp
from jax import lax
from jax.experimental import pallas as pl
from jax.experimental.pallas import tpu as pltpu
from jax.experimental.pallas import tpu_sc as plsc

N_NODE = 10000
E = 320000
D = 16
D_IN = 128
NS = 16
EPT = E // NS
CHUNK = 2000
N_CHUNKS = EPT // CHUNK
ROWS_A = 640
ROWS_B = N_NODE - 15 * ROWS_A
EPT2 = EPT // 2
HSUB = 80

_MESH = plsc.VectorSubcoreMesh(
    core_axis_name="c", subcore_axis_name="s", num_cores=2, num_subcores=NS
)
_NO_TC_TILING = pltpu.CompilerParams(use_tc_tiling_on_sc=False)

_F32 = jnp.float32
_VEC = jax.ShapeDtypeStruct((N_NODE,), _F32)
_TAB = jax.ShapeDtypeStruct((N_NODE, D), _F32)


def _rsqrt16v(d):
    k = jnp.zeros((16,), jnp.int32)
    for j in range(1, 10):
        k = k + jnp.where(d >= float(4 ** j), 1, 0)
    thv = jnp.full((16,), 1.0, _F32)
    for j in range(1, 10):
        thv = jnp.where(k >= j, thv * 0.5, thv)
    m = d * thv * thv
    y = 1.2 - 0.18 * m
    for _ in range(3):
        y = y * (1.5 - 0.5 * m * y * y)
    return jnp.where(d > 0.5, y * thv, 0.0)


_GDIMS = lax.GatherDimensionNumbers(
    offset_dims=(), collapsed_slice_dims=(0,), start_index_map=(0,))


def _splat(vec, j):
    idx = jnp.full((16, 1), j, jnp.int32)
    return lax.gather(vec, idx, _GDIMS, slice_sizes=(1,),
                      mode=lax.GatherScatterMode.PROMISE_IN_BOUNDS)


@functools.partial(
    pl.kernel,
    out_type=(_VEC, _VEC, _VEC, _VEC),
    mesh=_MESH,
    compiler_params=_NO_TC_TILING,
    scratch_types=[
        pltpu.VMEM((4, EPT2), jnp.int32),
        pltpu.VMEM((EPT2,), _F32),
        pltpu.VMEM_SHARED((N_NODE,), _F32),
        pltpu.VMEM_SHARED((N_NODE,), _F32),
        pltpu.SemaphoreType.DMA,
        pltpu.SemaphoreType.DMA,
        pltpu.SemaphoreType.DMA,
        pltpu.SemaphoreType.DMA,
    ],
)
def _hist_kernel(edges0, edges1, zeros_hbm, ones_hbm,
                 deg_s0, deg_d0, deg_s1, deg_d1,
                 idx4, ones_v, hist_r_sp, hist_c_sp, sa, sb, sc_, sd):
    c = lax.axis_index("c")
    s = lax.axis_index("s")

    @pl.when(s == 0)
    def _():
        pltpu.sync_copy(zeros_hbm, hist_r_sp)
        pltpu.sync_copy(zeros_hbm, hist_c_sp)

    pltpu.sync_copy(ones_hbm, ones_v)
    plsc.subcore_barrier()

    def do_rel(r, e_hbm):
        @pl.when(c == r)
        def _():
            base = s * EPT
            pltpu.sync_copy(e_hbm.at[0, pl.ds(base, EPT2)], idx4.at[0])
            ca = pltpu.async_copy(ones_v, hist_r_sp.at[idx4.at[0]], sa,
                                  add=True)
            pltpu.sync_copy(e_hbm.at[0, pl.ds(base + EPT2, EPT2)], idx4.at[1])
            cb = pltpu.async_copy(ones_v, hist_r_sp.at[idx4.at[1]], sb,
                                  add=True)
            pltpu.sync_copy(e_hbm.at[1, pl.ds(base, EPT2)], idx4.at[2])
            cc = pltpu.async_copy(ones_v, hist_c_sp.at[idx4.at[2]], sc_,
                                  add=True)
            pltpu.sync_copy(e_hbm.at[1, pl.ds(base + EPT2, EPT2)], idx4.at[3])
            cd = pltpu.async_copy(ones_v, hist_c_sp.at[idx4.at[3]], sd,
                                  add=True)
            ca.wait()
            cb.wait()
            cc.wait()
            cd.wait()

    do_rel(0, edges0)
    do_rel(1, edges1)
    plsc.subcore_barrier()

    @pl.when(s == 0)
    def _():
        @pl.when(c == 0)
        def _():
            pltpu.sync_copy(hist_r_sp, deg_s0)
            pltpu.sync_copy(hist_c_sp, deg_d0)

        @pl.when(c == 1)
        def _():
            pltpu.sync_copy(hist_r_sp, deg_s1)
            pltpu.sync_copy(hist_c_sp, deg_d1)


@functools.partial(
    pl.kernel,
    out_type=(_TAB, _TAB, _TAB, _TAB),
    mesh=_MESH,
    compiler_params=_NO_TC_TILING,
    scratch_types=[
        pltpu.VMEM((2, CHUNK), jnp.int32),
        pltpu.VMEM((2, CHUNK), jnp.int32),
        pltpu.VMEM((2, CHUNK, D), _F32),
        pltpu.VMEM((ROWS_A, D), _F32),
        pltpu.VMEM((HSUB, D_IN), _F32),
        pltpu.VMEM((N_NODE,), _F32),
        pltpu.VMEM((D,), _F32),
        pltpu.VMEM_SHARED((N_NODE, D), _F32),
        pltpu.SemaphoreType.DMA,
        pltpu.SemaphoreType.DMA,
    ],
)
def _main_kernel(h0, h1, edges0, edges1,
                 deg_s0, deg_d0, deg_s1, deg_d1, bias0, bias1, zeros_hbm,
                 out0, out1, g0, g1,
                 rid_v, cid_v, rows_v, rbuf, hbuf, dv, b_v, acc_sp,
                 sem0, sem1):
    c = lax.axis_index("c")
    s = lax.axis_index("s")
    sems = (sem0, sem1)

    pltpu.sync_copy(zeros_hbm.at[pl.ds(s * (N_NODE // NS), N_NODE // NS)],
                    acc_sp.at[pl.ds(s * (N_NODE // NS), N_NODE // NS)])

    def scale_rows(base, nrows, add_bias):
        bvec = b_v[...]

        def grp(g, carry):
            off = pl.multiple_of(base + g * 16, 16)
            scale = _rsqrt16v(dv[pl.ds(off, 16)])
            for j in range(16):
                i = g * 16 + j
                if add_bias:
                    rbuf[i] = rbuf[i] * _splat(scale, j) + bvec
                else:
                    rbuf[i] = rbuf[i] * _splat(scale, j)
            return carry

        lax.fori_loop(0, nrows // 16, grp, 0)

    def prepass(base, nrows, h_hbm, g_hbm):
        pltpu.sync_copy(h_hbm.at[pl.ds(base, nrows)], rbuf.at[pl.ds(0, nrows)])
        scale_rows(base, nrows, add_bias=False)
        pltpu.sync_copy(rbuf.at[pl.ds(0, nrows)], g_hbm.at[pl.ds(base, nrows)])

    def epilogue(base, nrows, out_hbm):
        pltpu.sync_copy(acc_sp.at[pl.ds(base, nrows)],
                        rbuf.at[pl.ds(0, nrows)])
        scale_rows(base, nrows, add_bias=True)
        pltpu.sync_copy(rbuf.at[pl.ds(0, nrows)],
                        out_hbm.at[pl.ds(base, nrows)])

    def by_tile(fn):
        @pl.when(s < NS - 1)
        def _():
            fn(s * ROWS_A, ROWS_A)

        @pl.when(s == NS - 1)
        def _():
            fn((NS - 1) * ROWS_A, ROWS_B)

    def do_rel(r, h_hbm, g_hbm, e_hbm, ds_hbm, dd_hbm, bias_hbm):
        @pl.when(c == r)
        def _():
            pltpu.sync_copy(ds_hbm, dv)
            by_tile(lambda base, n: prepass(base, n, h_hbm, g_hbm))
            pltpu.sync_copy(dd_hbm, dv)
            pltpu.sync_copy(bias_hbm, b_v)
            plsc.subcore_barrier()

            base = s * EPT
            pltpu.sync_copy(e_hbm.at[0, pl.ds(base, CHUNK)], rid_v.at[0])
            pltpu.sync_copy(e_hbm.at[1, pl.ds(base, CHUNK)], cid_v.at[0])
            pltpu.async_copy(g_hbm.at[rid_v.at[0]], rows_v.at[0], sems[0])
            for k in range(N_CHUNKS):
                cur = k % 2
                nxt = 1 - cur
                if k + 1 < N_CHUNKS:
                    off = base + (k + 1) * CHUNK
                    pltpu.sync_copy(e_hbm.at[0, pl.ds(off, CHUNK)],
                                    rid_v.at[nxt])
                    pltpu.sync_copy(e_hbm.at[1, pl.ds(off, CHUNK)],
                                    cid_v.at[nxt])
                    pltpu.async_copy(g_hbm.at[rid_v.at[nxt]], rows_v.at[nxt],
                                     sems[nxt])
                pltpu.make_async_copy(g_hbm.at[rid_v.at[cur]],
                                      rows_v.at[cur], sems[cur]).wait()
                pltpu.sync_copy(rows_v.at[cur], acc_sp.at[cid_v.at[cur]],
                                add=True)

    do_rel(0, h0, g0, edges0, deg_s0, deg_d0, bias0)
    do_rel(1, h1, g1, edges1, deg_s1, deg_d1, bias1)
    plsc.subcore_barrier()

    def fin_rel(r, out_hbm):
        @pl.when(c == r)
        def _():
            by_tile(lambda base, n: epilogue(base, n, out_hbm))

    fin_rel(0, out0)
    fin_rel(1, out1)


def _rsqrt_body(a_ref, b_ref, c_ref, d_ref, oa_ref, ob_ref, oc_ref, od_ref):
    for i_ref, o_ref in ((a_ref, oa_ref), (b_ref, ob_ref),
                         (c_ref, oc_ref), (d_ref, od_ref)):
        deg = i_ref[...]
        o_ref[...] = jnp.where(deg > 0, lax.rsqrt(jnp.maximum(deg, 1e-12)),
                               0.0)


def _tc_rsqrt(a, b, c, d):
    return pl.pallas_call(
        _rsqrt_body,
        out_shape=(_VEC, _VEC, _VEC, _VEC),
    )(a, b, c, d)


def _matmul_body(xa_ref, xt_ref, wi_ref, wr_ref, h0_ref, h1_ref):
    h0_ref[...] = jnp.dot(xa_ref[...], wi_ref[...],
                          preferred_element_type=_F32)
    h1_ref[...] = jnp.dot(xt_ref[...], wr_ref[...],
                          preferred_element_type=_F32)


def _tc_matmul(x_account, x_transaction, W_init, W_recv):
    return pl.pallas_call(
        _matmul_body,
        out_shape=(jax.ShapeDtypeStruct((N_NODE, D), _F32),
                   jax.ShapeDtypeStruct((N_NODE, D), _F32)),
    )(x_account, x_transaction, W_init, W_recv)


def kernel(x_account, x_transaction, W_init, b_init, W_recv, b_recv,
           edge_index_initiates, edge_index_receives):
    edges0 = edge_index_initiates.astype(jnp.int32)
    edges1 = edge_index_receives.astype(jnp.int32)
    zeros_n = jnp.zeros((N_NODE,), _F32)
    ones_e = jnp.ones((EPT2,), _F32)
    zeros_s = jnp.zeros((N_NODE, D), _F32)

    deg_s0, deg_d0, deg_s1, deg_d1 = _hist_kernel(
        edges0, edges1, zeros_n, ones_e)
    h0, h1 = _tc_matmul(x_account, x_transaction, W_init, W_recv)
    out0, out1, _, _ = _main_kernel(
        h0, h1, edges0, edges1,
        deg_s0, deg_d0, deg_s1, deg_d1, b_init, b_recv, zeros_s)
    return (out1, out0)


# speedup vs baseline: 1.0452x; 1.0452x over previous
import functools

import jax
import jax.numpy as jnp
from jax import lax
from jax.experimental import pallas as pl
from jax.experimental.pallas import tpu as pltpu
from jax.experimental.pallas import tpu_sc as plsc

N_NODE = 10000
E = 320000
D = 16
D_IN = 128
NS = 16
EPT = E // NS
CHUNK = 2000
N_CHUNKS = EPT // CHUNK
ROWS_A = 640
ROWS_B = N_NODE - 15 * ROWS_A
EPT2 = EPT // 2

_MESH = plsc.VectorSubcoreMesh(
    core_axis_name="c", subcore_axis_name="s", num_cores=2, num_subcores=NS
)
_NO_TC_TILING = pltpu.CompilerParams(use_tc_tiling_on_sc=False)

_F32 = jnp.float32
_VEC = jax.ShapeDtypeStruct((N_NODE,), _F32)
_TAB = jax.ShapeDtypeStruct((N_NODE, D), _F32)


_GDIMS = lax.GatherDimensionNumbers(
    offset_dims=(), collapsed_slice_dims=(0,), start_index_map=(0,))


def _splat(vec, j):
    idx = jnp.full((16, 1), j, jnp.int32)
    return lax.gather(vec, idx, _GDIMS, slice_sizes=(1,),
                      mode=lax.GatherScatterMode.PROMISE_IN_BOUNDS)


@functools.partial(
    pl.kernel,
    out_type=(_VEC, _VEC, _VEC, _VEC),
    mesh=_MESH,
    compiler_params=_NO_TC_TILING,
    scratch_types=[
        pltpu.VMEM((4, EPT2), jnp.int32),
        pltpu.VMEM((EPT2,), _F32),
        pltpu.VMEM_SHARED((N_NODE,), _F32),
        pltpu.VMEM_SHARED((N_NODE,), _F32),
        pltpu.SemaphoreType.DMA,
        pltpu.SemaphoreType.DMA,
        pltpu.SemaphoreType.DMA,
        pltpu.SemaphoreType.DMA,
    ],
)
def _hist_kernel(edges0, edges1, zeros_hbm, ones_hbm,
                 deg_s0, deg_d0, deg_s1, deg_d1,
                 idx4, ones_v, hist_r_sp, hist_c_sp, sa, sb, sc_, sd):
    c = lax.axis_index("c")
    s = lax.axis_index("s")

    @pl.when(s == 0)
    def _():
        pltpu.sync_copy(zeros_hbm, hist_r_sp)
        pltpu.sync_copy(zeros_hbm, hist_c_sp)

    pltpu.sync_copy(ones_hbm, ones_v)
    plsc.subcore_barrier()

    def do_rel(r, e_hbm):
        @pl.when(c == r)
        def _():
            base = s * EPT
            pltpu.sync_copy(e_hbm.at[0, pl.ds(base, EPT2)], idx4.at[0])
            ca = pltpu.async_copy(ones_v, hist_r_sp.at[idx4.at[0]], sa,
                                  add=True)
            pltpu.sync_copy(e_hbm.at[0, pl.ds(base + EPT2, EPT2)], idx4.at[1])
            cb = pltpu.async_copy(ones_v, hist_r_sp.at[idx4.at[1]], sb,
                                  add=True)
            pltpu.sync_copy(e_hbm.at[1, pl.ds(base, EPT2)], idx4.at[2])
            cc = pltpu.async_copy(ones_v, hist_c_sp.at[idx4.at[2]], sc_,
                                  add=True)
            pltpu.sync_copy(e_hbm.at[1, pl.ds(base + EPT2, EPT2)], idx4.at[3])
            cd = pltpu.async_copy(ones_v, hist_c_sp.at[idx4.at[3]], sd,
                                  add=True)
            ca.wait()
            cb.wait()
            cc.wait()
            cd.wait()

    do_rel(0, edges0)
    do_rel(1, edges1)
    plsc.subcore_barrier()

    @pl.when(s == 0)
    def _():
        @pl.when(c == 0)
        def _():
            pltpu.sync_copy(hist_r_sp, deg_s0)
            pltpu.sync_copy(hist_c_sp, deg_d0)

        @pl.when(c == 1)
        def _():
            pltpu.sync_copy(hist_r_sp, deg_s1)
            pltpu.sync_copy(hist_c_sp, deg_d1)


@functools.partial(
    pl.kernel,
    out_type=(_TAB, _TAB, _TAB, _TAB),
    mesh=_MESH,
    compiler_params=_NO_TC_TILING,
    scratch_types=[
        pltpu.VMEM((2, CHUNK), jnp.int32),
        pltpu.VMEM((2, CHUNK), jnp.int32),
        pltpu.VMEM((2, CHUNK, D), _F32),
        pltpu.VMEM((ROWS_A, D), _F32),
        pltpu.VMEM((N_NODE,), _F32),
        pltpu.VMEM((D,), _F32),
        pltpu.VMEM_SHARED((N_NODE, D), _F32),
        pltpu.SemaphoreType.DMA,
        pltpu.SemaphoreType.DMA,
    ],
)
def _main_kernel(h0, h1, edges0, edges1,
                 deg_s0, deg_d0, deg_s1, deg_d1, bias0, bias1, zeros_hbm,
                 out0, out1, g0, g1,
                 rid_v, cid_v, rows_v, rbuf, dv, b_v, acc_sp,
                 sem0, sem1):
    c = lax.axis_index("c")
    s = lax.axis_index("s")
    sems = (sem0, sem1)

    pltpu.sync_copy(zeros_hbm.at[pl.ds(s * (N_NODE // NS), N_NODE // NS)],
                    acc_sp.at[pl.ds(s * (N_NODE // NS), N_NODE // NS)])

    def scale_rows(base, nrows, add_bias):
        bvec = b_v[...]

        def grp(g, carry):
            off = pl.multiple_of(base + g * 16, 16)
            scale = dv[pl.ds(off, 16)]
            for j in range(16):
                i = g * 16 + j
                if add_bias:
                    rbuf[i] = rbuf[i] * _splat(scale, j) + bvec
                else:
                    rbuf[i] = rbuf[i] * _splat(scale, j)
            return carry

        lax.fori_loop(0, nrows // 16, grp, 0)

    def prepass(base, nrows, h_hbm, g_hbm):
        pltpu.sync_copy(h_hbm.at[pl.ds(base, nrows)], rbuf.at[pl.ds(0, nrows)])
        scale_rows(base, nrows, add_bias=False)
        pltpu.sync_copy(rbuf.at[pl.ds(0, nrows)], g_hbm.at[pl.ds(base, nrows)])

    def epilogue(base, nrows, out_hbm):
        pltpu.sync_copy(acc_sp.at[pl.ds(base, nrows)],
                        rbuf.at[pl.ds(0, nrows)])
        scale_rows(base, nrows, add_bias=True)
        pltpu.sync_copy(rbuf.at[pl.ds(0, nrows)],
                        out_hbm.at[pl.ds(base, nrows)])

    def by_tile(fn):
        @pl.when(s < NS - 1)
        def _():
            fn(s * ROWS_A, ROWS_A)

        @pl.when(s == NS - 1)
        def _():
            fn((NS - 1) * ROWS_A, ROWS_B)

    def do_rel(r, h_hbm, g_hbm, e_hbm, ds_hbm, dd_hbm, bias_hbm):
        @pl.when(c == r)
        def _():
            pltpu.sync_copy(ds_hbm, dv)
            by_tile(lambda base, n: prepass(base, n, h_hbm, g_hbm))
            pltpu.sync_copy(dd_hbm, dv)
            pltpu.sync_copy(bias_hbm, b_v)
            plsc.subcore_barrier()

            base = s * EPT
            pltpu.sync_copy(e_hbm.at[0, pl.ds(base, CHUNK)], rid_v.at[0])
            pltpu.sync_copy(e_hbm.at[1, pl.ds(base, CHUNK)], cid_v.at[0])
            pltpu.async_copy(g_hbm.at[rid_v.at[0]], rows_v.at[0], sems[0])
            for k in range(N_CHUNKS):
                cur = k % 2
                nxt = 1 - cur
                if k + 1 < N_CHUNKS:
                    off = base + (k + 1) * CHUNK
                    pltpu.sync_copy(e_hbm.at[0, pl.ds(off, CHUNK)],
                                    rid_v.at[nxt])
                    pltpu.sync_copy(e_hbm.at[1, pl.ds(off, CHUNK)],
                                    cid_v.at[nxt])
                    pltpu.async_copy(g_hbm.at[rid_v.at[nxt]], rows_v.at[nxt],
                                     sems[nxt])
                pltpu.make_async_copy(g_hbm.at[rid_v.at[cur]],
                                      rows_v.at[cur], sems[cur]).wait()
                pltpu.sync_copy(rows_v.at[cur], acc_sp.at[cid_v.at[cur]],
                                add=True)

    do_rel(0, h0, g0, edges0, deg_s0, deg_d0, bias0)
    do_rel(1, h1, g1, edges1, deg_s1, deg_d1, bias1)
    plsc.subcore_barrier()

    def fin_rel(r, out_hbm):
        @pl.when(c == r)
        def _():
            by_tile(lambda base, n: epilogue(base, n, out_hbm))

    fin_rel(0, out0)
    fin_rel(1, out1)


def _rsqrt_body(a_ref, b_ref, c_ref, d_ref, oa_ref, ob_ref, oc_ref, od_ref):
    for i_ref, o_ref in ((a_ref, oa_ref), (b_ref, ob_ref),
                         (c_ref, oc_ref), (d_ref, od_ref)):
        deg = i_ref[...]
        o_ref[...] = jnp.where(deg > 0, lax.rsqrt(jnp.maximum(deg, 1e-12)),
                               0.0)


def _tc_rsqrt(a, b, c, d):
    return pl.pallas_call(
        _rsqrt_body,
        out_shape=(_VEC, _VEC, _VEC, _VEC),
    )(a, b, c, d)


def _matmul_body(xa_ref, xt_ref, wi_ref, wr_ref, h0_ref, h1_ref):
    h0_ref[...] = jnp.dot(xa_ref[...], wi_ref[...],
                          preferred_element_type=_F32)
    h1_ref[...] = jnp.dot(xt_ref[...], wr_ref[...],
                          preferred_element_type=_F32)


def _tc_matmul(x_account, x_transaction, W_init, W_recv):
    return pl.pallas_call(
        _matmul_body,
        out_shape=(jax.ShapeDtypeStruct((N_NODE, D), _F32),
                   jax.ShapeDtypeStruct((N_NODE, D), _F32)),
    )(x_account, x_transaction, W_init, W_recv)


def kernel(x_account, x_transaction, W_init, b_init, W_recv, b_recv,
           edge_index_initiates, edge_index_receives):
    edges0 = edge_index_initiates.astype(jnp.int32)
    edges1 = edge_index_receives.astype(jnp.int32)
    zeros_n = jnp.zeros((N_NODE,), _F32)
    ones_e = jnp.ones((EPT2,), _F32)
    zeros_s = jnp.zeros((N_NODE, D), _F32)

    deg_s0, deg_d0, deg_s1, deg_d1 = _hist_kernel(
        edges0, edges1, zeros_n, ones_e)
    h0, h1 = _tc_matmul(x_account, x_transaction, W_init, W_recv)
    rs0, rd0, rs1, rd1 = _tc_rsqrt(deg_s0, deg_d0, deg_s1, deg_d1)
    out0, out1, _, _ = _main_kernel(
        h0, h1, edges0, edges1,
        rs0, rd0, rs1, rd1, b_init, b_recv, zeros_s)
    return (out1, out0)
